# Initial kernel scaffold; baseline (speedup 1.0000x reference)
import jax
import jax.numpy as jnp
from jax.experimental import pallas as pl


def kernel(src, emb1_weight, emb2_weight):
    raise NotImplementedError("write your pallas kernel here")


# speedup vs baseline: 2.6462x; 2.6462x over previous
import functools

import jax
import jax.numpy as jnp
from jax import lax
from jax.experimental import pallas as pl
from jax.experimental.pallas import tpu as pltpu
from jax.experimental.pallas import tpu_sc as plsc

INPUT_DIM = 100000
LAYER1_DIM = 64
HID_DIM = 128
BATCH = 4096
HIST = 50

NTOK = BATCH * HIST
NW = 32
PER_W = NTOK // NW
CHUNK = 128
NCH = PER_W // CHUNK

ROWS_BLK = 2000


def _proj_body(w1_ref, w2_ref, out_ref):
    out_ref[...] = lax.dot_general(
        w1_ref[...], w2_ref[...],
        dimension_numbers=(((1,), (1,)), ((), ())),
        preferred_element_type=jnp.float32,
    )


def _project(emb1_weight, emb2_weight):
    return pl.pallas_call(
        _proj_body,
        grid=(INPUT_DIM // ROWS_BLK,),
        in_specs=[
            pl.BlockSpec((ROWS_BLK, LAYER1_DIM), lambda i: (i, 0)),
            pl.BlockSpec((HID_DIM, LAYER1_DIM), lambda i: (0, 0)),
        ],
        out_specs=pl.BlockSpec((ROWS_BLK, HID_DIM), lambda i: (i, 0)),
        out_shape=jax.ShapeDtypeStruct((INPUT_DIM, HID_DIM), jnp.float32),
    )(emb1_weight, emb2_weight)


@functools.lru_cache(maxsize=1)
def _build_gather():
    mesh = plsc.VectorSubcoreMesh(core_axis_name="c", subcore_axis_name="s")

    @functools.partial(
        pl.kernel,
        mesh=mesh,
        out_type=jax.ShapeDtypeStruct((NTOK, HID_DIM), jnp.float32),
        scratch_types=[
            pltpu.VMEM((NCH, CHUNK), jnp.int32),
            pltpu.VMEM((CHUNK, HID_DIM), jnp.float32),
            pltpu.SemaphoreType.DMA,
        ],
    )
    def _gather(proj_hbm, idx_hbm, out_hbm, idx_v, buf, sem):
        wid = lax.axis_index("s") * 2 + lax.axis_index("c")
        base = wid * PER_W
        pltpu.sync_copy(idx_hbm.at[wid], idx_v)

        def body(j, carry):
            pltpu.async_copy(proj_hbm.at[idx_v.at[j]], buf, sem).wait()
            pltpu.sync_copy(buf, out_hbm.at[pl.ds(base + j * CHUNK, CHUNK)])
            return carry

        lax.fori_loop(0, NCH, body, jnp.int32(0))

    return _gather


def kernel(src, emb1_weight, emb2_weight):
    proj = _project(emb1_weight, emb2_weight)
    idx = src.reshape(NW, NCH, CHUNK)
    out = _build_gather()(proj, idx)
    return out.reshape(BATCH, HIST, HID_DIM)


# speedup vs baseline: 2.9004x; 1.0961x over previous
import functools

import jax
import jax.numpy as jnp
from jax import lax
from jax.experimental import pallas as pl
from jax.experimental.pallas import tpu as pltpu
from jax.experimental.pallas import tpu_sc as plsc

INPUT_DIM = 100000
LAYER1_DIM = 64
HID_DIM = 128
BATCH = 4096
HIST = 50

NTOK = BATCH * HIST
NW = 32
PER_W = NTOK // NW
CHUNK = 128
NCH = PER_W // CHUNK

ROWS_BLK = 2000


def _proj_body(w1_ref, w2_ref, out_ref):
    out_ref[...] = lax.dot_general(
        w1_ref[...], w2_ref[...],
        dimension_numbers=(((1,), (1,)), ((), ())),
        preferred_element_type=jnp.float32,
    )


def _project(emb1_weight, emb2_weight):
    return pl.pallas_call(
        _proj_body,
        grid=(INPUT_DIM // ROWS_BLK,),
        in_specs=[
            pl.BlockSpec((ROWS_BLK, LAYER1_DIM), lambda i: (i, 0)),
            pl.BlockSpec((HID_DIM, LAYER1_DIM), lambda i: (0, 0)),
        ],
        out_specs=pl.BlockSpec((ROWS_BLK, HID_DIM), lambda i: (i, 0)),
        out_shape=jax.ShapeDtypeStruct((INPUT_DIM, HID_DIM), jnp.float32),
    )(emb1_weight, emb2_weight)


NBUF = 5


@functools.lru_cache(maxsize=1)
def _build_gather():
    mesh = plsc.VectorSubcoreMesh(core_axis_name="c", subcore_axis_name="s")

    @functools.partial(
        pl.kernel,
        mesh=mesh,
        out_type=jax.ShapeDtypeStruct((NTOK, HID_DIM), jnp.float32),
        scratch_types=[
            pltpu.VMEM((NCH, CHUNK), jnp.int32),
            *[pltpu.VMEM((CHUNK, HID_DIM), jnp.float32) for _ in range(NBUF)],
            *[pltpu.SemaphoreType.DMA for _ in range(2 * NBUF)],
        ],
    )
    def _gather(proj_hbm, idx_hbm, out_hbm, idx_v, *scratch):
        bufs = scratch[:NBUF]
        gsems = scratch[NBUF:2 * NBUF]
        wsems = scratch[2 * NBUF:]
        wid = lax.axis_index("s") * 2 + lax.axis_index("c")
        base = wid * PER_W
        pltpu.sync_copy(idx_hbm.at[wid], idx_v)

        for b in range(NBUF):
            pltpu.async_copy(proj_hbm.at[idx_v.at[b]], bufs[b], gsems[b])

        def body(t, carry):
            i = t * NBUF
            for b in range(NBUF):
                j = i + b
                dst = out_hbm.at[pl.ds(base + j * CHUNK, CHUNK)]
                pltpu.make_async_copy(proj_hbm.at[idx_v.at[j]], bufs[b], gsems[b]).wait()
                pltpu.async_copy(bufs[b], dst, wsems[b])

                @pl.when(j + NBUF < NCH)
                def _():
                    pltpu.make_async_copy(bufs[b], dst, wsems[b]).wait()
                    pltpu.async_copy(proj_hbm.at[idx_v.at[j + NBUF]], bufs[b], gsems[b])

            return carry

        lax.fori_loop(0, NCH // NBUF, body, jnp.int32(0))

        for b in range(NBUF):
            j = NCH - NBUF + b
            pltpu.make_async_copy(
                bufs[b], out_hbm.at[pl.ds(base + j * CHUNK, CHUNK)], wsems[b]
            ).wait()

    return _gather


def kernel(src, emb1_weight, emb2_weight):
    proj = _project(emb1_weight, emb2_weight)
    idx = src.reshape(NW, NCH, CHUNK)
    out = _build_gather()(proj, idx)
    return out.reshape(BATCH, HIST, HID_DIM)


# speedup vs baseline: 2.9047x; 1.0015x over previous
import functools

import jax
import jax.numpy as jnp
from jax import lax
from jax.experimental import pallas as pl
from jax.experimental.pallas import tpu as pltpu
from jax.experimental.pallas import tpu_sc as plsc

INPUT_DIM = 100000
LAYER1_DIM = 64
HID_DIM = 128
BATCH = 4096
HIST = 50

NTOK = BATCH * HIST
NW = 32
PER_W = NTOK // NW
CHUNK = 128
NCH = PER_W // CHUNK

ROWS_BLK = 2000


def _proj_body(w1_ref, w2_ref, out_ref):
    out_ref[...] = lax.dot_general(
        w1_ref[...], w2_ref[...],
        dimension_numbers=(((1,), (1,)), ((), ())),
        preferred_element_type=jnp.float32,
    )


def _project(emb1_weight, emb2_weight):
    return pl.pallas_call(
        _proj_body,
        grid=(INPUT_DIM // ROWS_BLK,),
        in_specs=[
            pl.BlockSpec((ROWS_BLK, LAYER1_DIM), lambda i: (i, 0)),
            pl.BlockSpec((HID_DIM, LAYER1_DIM), lambda i: (0, 0)),
        ],
        out_specs=pl.BlockSpec((ROWS_BLK, HID_DIM), lambda i: (i, 0)),
        out_shape=jax.ShapeDtypeStruct((INPUT_DIM, HID_DIM), jnp.float32),
    )(emb1_weight, emb2_weight)


NBUF = 5


@functools.lru_cache(maxsize=1)
def _build_gather():
    mesh = plsc.VectorSubcoreMesh(core_axis_name="c", subcore_axis_name="s")

    @functools.partial(
        pl.kernel,
        mesh=mesh,
        compiler_params=pltpu.CompilerParams(use_tc_tiling_on_sc=True),
        out_type=jax.ShapeDtypeStruct((NTOK, HID_DIM), jnp.float32),
        scratch_types=[
            pltpu.VMEM((NCH, CHUNK), jnp.int32),
            *[pltpu.VMEM((CHUNK, HID_DIM), jnp.float32) for _ in range(NBUF)],
            *[pltpu.SemaphoreType.DMA for _ in range(2 * NBUF)],
        ],
    )
    def _gather(proj_hbm, idx_hbm, out_hbm, idx_v, *scratch):
        bufs = scratch[:NBUF]
        gsems = scratch[NBUF:2 * NBUF]
        wsems = scratch[2 * NBUF:]
        wid = lax.axis_index("s") * 2 + lax.axis_index("c")
        base = wid * PER_W
        pltpu.sync_copy(idx_hbm.at[wid], idx_v)

        for b in range(NBUF):
            pltpu.async_copy(proj_hbm.at[idx_v.at[b]], bufs[b], gsems[b])

        def body(t, carry):
            i = t * NBUF
            for b in range(NBUF):
                j = i + b
                dst = out_hbm.at[pl.ds(base + j * CHUNK, CHUNK)]
                pltpu.make_async_copy(proj_hbm.at[idx_v.at[j]], bufs[b], gsems[b]).wait()
                pltpu.async_copy(bufs[b], dst, wsems[b])

                @pl.when(j + NBUF < NCH)
                def _():
                    pltpu.make_async_copy(bufs[b], dst, wsems[b]).wait()
                    pltpu.async_copy(proj_hbm.at[idx_v.at[j + NBUF]], bufs[b], gsems[b])

            return carry

        lax.fori_loop(0, NCH // NBUF, body, jnp.int32(0))

        for b in range(NBUF):
            j = NCH - NBUF + b
            pltpu.make_async_copy(
                bufs[b], out_hbm.at[pl.ds(base + j * CHUNK, CHUNK)], wsems[b]
            ).wait()

    return _gather


def kernel(src, emb1_weight, emb2_weight):
    proj = _project(emb1_weight, emb2_weight)
    idx = src.reshape(NW, NCH, CHUNK)
    out = _build_gather()(proj, idx)
    return out.reshape(BATCH, HIST, HID_DIM)


# speedup vs baseline: 4.3562x; 1.4997x over previous
import functools

import jax
import jax.numpy as jnp
from jax import lax
from jax.experimental import pallas as pl
from jax.experimental.pallas import tpu as pltpu
from jax.experimental.pallas import tpu_sc as plsc

INPUT_DIM = 100000
LAYER1_DIM = 64
HID_DIM = 128
BATCH = 4096
HIST = 50

NW = 32
BPW = BATCH // NW
NBUF = 8

ROWS_BLK = 2000


def _proj_body(w1_ref, w2_ref, out_ref):
    out_ref[...] = lax.dot_general(
        w1_ref[...], w2_ref[...],
        dimension_numbers=(((1,), (1,)), ((), ())),
        preferred_element_type=jnp.float32,
    )


def _project(emb1_weight, emb2_weight):
    return pl.pallas_call(
        _proj_body,
        grid=(INPUT_DIM // ROWS_BLK,),
        in_specs=[
            pl.BlockSpec((ROWS_BLK, LAYER1_DIM), lambda i: (i, 0)),
            pl.BlockSpec((HID_DIM, LAYER1_DIM), lambda i: (0, 0)),
        ],
        out_specs=pl.BlockSpec((ROWS_BLK, HID_DIM), lambda i: (i, 0)),
        out_shape=jax.ShapeDtypeStruct((INPUT_DIM, HID_DIM), jnp.float32),
    )(emb1_weight, emb2_weight)


@functools.lru_cache(maxsize=1)
def _build_gather():
    mesh = plsc.VectorSubcoreMesh(core_axis_name="c", subcore_axis_name="s")

    @functools.partial(
        pl.kernel,
        mesh=mesh,
        out_type=jax.ShapeDtypeStruct((BATCH, HIST, HID_DIM), jnp.float32),
        scratch_types=[
            pltpu.VMEM((BPW, HIST), jnp.int32),
            *[pltpu.VMEM((HIST, HID_DIM), jnp.float32) for _ in range(NBUF)],
            *[pltpu.SemaphoreType.DMA for _ in range(2 * NBUF)],
        ],
    )
    def _gather(proj_hbm, idx_hbm, out_hbm, idx_v, *scratch):
        bufs = scratch[:NBUF]
        gsems = scratch[NBUF:2 * NBUF]
        wsems = scratch[2 * NBUF:]
        wid = lax.axis_index("s") * 2 + lax.axis_index("c")
        base = wid * BPW
        pltpu.sync_copy(idx_hbm.at[wid], idx_v)

        for b in range(NBUF):
            pltpu.async_copy(proj_hbm.at[idx_v.at[b]], bufs[b], gsems[b])

        def body(t, carry):
            i = t * NBUF
            for b in range(NBUF):
                j = i + b
                dst = out_hbm.at[base + j]
                pltpu.make_async_copy(proj_hbm.at[idx_v.at[j]], bufs[b], gsems[b]).wait()
                pltpu.async_copy(bufs[b], dst, wsems[b])

                @pl.when(j + NBUF < BPW)
                def _():
                    pltpu.make_async_copy(bufs[b], dst, wsems[b]).wait()
                    pltpu.async_copy(proj_hbm.at[idx_v.at[j + NBUF]], bufs[b], gsems[b])

            return carry

        lax.fori_loop(0, BPW // NBUF, body, jnp.int32(0))

        for b in range(NBUF):
            j = BPW - NBUF + b
            pltpu.make_async_copy(bufs[b], out_hbm.at[base + j], wsems[b]).wait()

    return _gather


def kernel(src, emb1_weight, emb2_weight):
    proj = _project(emb1_weight, emb2_weight)
    idx = src.reshape(NW, BPW, HIST)
    return _build_gather()(proj, idx)


# speedup vs baseline: 7.8462x; 1.8011x over previous
import functools

import jax
import jax.numpy as jnp
from jax import lax
from jax.experimental import pallas as pl
from jax.experimental.pallas import tpu as pltpu
from jax.experimental.pallas import tpu_sc as plsc

INPUT_DIM = 100000
LAYER1_DIM = 64
HID_DIM = 128
BATCH = 4096
HIST = 50

NW = 32
BPW = BATCH // NW
NCH = HIST
NBUF = 5

COLS_BLK = 2048


def _proj_body(w1t_ref, w2t_ref, out_ref):
    out_ref[...] = lax.dot_general(
        w1t_ref[...], w2t_ref[...],
        dimension_numbers=(((0,), (0,)), ((), ())),
        preferred_element_type=jnp.float32,
    )


def _project(emb1_t, emb2_t):
    return pl.pallas_call(
        _proj_body,
        grid=(pl.cdiv(INPUT_DIM, COLS_BLK),),
        in_specs=[
            pl.BlockSpec((LAYER1_DIM, COLS_BLK), lambda i: (0, i)),
            pl.BlockSpec((LAYER1_DIM, HID_DIM), lambda i: (0, 0)),
        ],
        out_specs=pl.BlockSpec((COLS_BLK, HID_DIM), lambda i: (i, 0)),
        out_shape=jax.ShapeDtypeStruct((INPUT_DIM, HID_DIM), jnp.float32),
    )(emb1_t, emb2_t)


@functools.lru_cache(maxsize=1)
def _build_gather():
    mesh = plsc.VectorSubcoreMesh(core_axis_name="c", subcore_axis_name="s")

    @functools.partial(
        pl.kernel,
        mesh=mesh,
        out_type=jax.ShapeDtypeStruct((HIST * NW, BPW, HID_DIM), jnp.float32),
        scratch_types=[
            pltpu.VMEM((NCH, BPW), jnp.int32),
            *[pltpu.VMEM((BPW, HID_DIM), jnp.float32) for _ in range(NBUF)],
            *[pltpu.SemaphoreType.DMA for _ in range(2 * NBUF)],
        ],
    )
    def _gather(proj_hbm, idx_hbm, out_hbm, idx_v, *scratch):
        bufs = scratch[:NBUF]
        gsems = scratch[NBUF:2 * NBUF]
        wsems = scratch[2 * NBUF:]
        wid = lax.axis_index("s") * 2 + lax.axis_index("c")
        pltpu.sync_copy(idx_hbm.at[wid], idx_v)

        for b in range(NBUF):
            pltpu.async_copy(proj_hbm.at[idx_v.at[b]], bufs[b], gsems[b])

        def body(t, carry):
            i = t * NBUF
            for b in range(NBUF):
                j = i + b
                dst = out_hbm.at[j * NW + wid]
                pltpu.make_async_copy(proj_hbm.at[idx_v.at[j]], bufs[b], gsems[b]).wait()
                pltpu.async_copy(bufs[b], dst, wsems[b])

                @pl.when(j + NBUF < NCH)
                def _():
                    pltpu.make_async_copy(bufs[b], dst, wsems[b]).wait()
                    pltpu.async_copy(proj_hbm.at[idx_v.at[j + NBUF]], bufs[b], gsems[b])

            return carry

        lax.fori_loop(0, NCH // NBUF, body, jnp.int32(0))

        for b in range(NBUF):
            j = NCH - NBUF + b
            pltpu.make_async_copy(bufs[b], out_hbm.at[j * NW + wid], wsems[b]).wait()

    return _gather


def kernel(src, emb1_weight, emb2_weight):
    proj = _project(emb1_weight.T, emb2_weight.T)
    idx = src.T.reshape(HIST, NW, BPW).transpose(1, 0, 2)
    out = _build_gather()(proj, idx)
    return out.reshape(HIST, BATCH, HID_DIM).transpose(1, 0, 2)


# speedup vs baseline: 8.6356x; 1.1006x over previous
import functools

import jax
import jax.numpy as jnp
from jax import lax
from jax.experimental import pallas as pl
from jax.experimental.pallas import tpu as pltpu
from jax.experimental.pallas import tpu_sc as plsc

INPUT_DIM = 100000
LAYER1_DIM = 64
HID_DIM = 128
BATCH = 4096
HIST = 50

NW = 32
BPW = BATCH // NW
NCH = HIST
NBUF = 5
NSPLIT = 2

COLS_BLK = 4096


def _proj_body(w1t_ref, w2t_ref, out_ref):
    out_ref[...] = lax.dot_general(
        w1t_ref[...], w2t_ref[...],
        dimension_numbers=(((0,), (0,)), ((), ())),
        preferred_element_type=jnp.float32,
    )


def _project(emb1_t, emb2_t):
    return pl.pallas_call(
        _proj_body,
        grid=(pl.cdiv(INPUT_DIM, COLS_BLK),),
        in_specs=[
            pl.BlockSpec((LAYER1_DIM, COLS_BLK), lambda i: (0, i)),
            pl.BlockSpec((LAYER1_DIM, HID_DIM), lambda i: (0, 0)),
        ],
        out_specs=pl.BlockSpec((COLS_BLK, HID_DIM), lambda i: (i, 0)),
        out_shape=jax.ShapeDtypeStruct((INPUT_DIM, HID_DIM), jnp.float32),
    )(emb1_t, emb2_t)


@functools.lru_cache(maxsize=1)
def _build_gather():
    mesh = plsc.VectorSubcoreMesh(core_axis_name="c", subcore_axis_name="s")

    @functools.partial(
        pl.kernel,
        mesh=mesh,
        out_type=jax.ShapeDtypeStruct((HIST * NW, BPW, HID_DIM), jnp.float32),
        scratch_types=[
            pltpu.VMEM((NCH, BPW), jnp.int32),
            *[pltpu.VMEM((BPW, HID_DIM), jnp.float32) for _ in range(NBUF)],
            *[pltpu.SemaphoreType.DMA for _ in range(2 * NBUF)],
        ],
    )
    def _gather(proj_hbm, idx_hbm, out_hbm, idx_v, *scratch):
        bufs = scratch[:NBUF]
        gsems = scratch[NBUF:2 * NBUF]
        wsems = scratch[2 * NBUF:]
        wid = lax.axis_index("s") * 2 + lax.axis_index("c")
        pltpu.sync_copy(idx_hbm.at[wid], idx_v)
        sub = BPW // NSPLIT

        def start_gather(j, b):
            for h in range(NSPLIT):
                pltpu.async_copy(
                    proj_hbm.at[idx_v.at[j, pl.ds(h * sub, sub)]],
                    bufs[b].at[pl.ds(h * sub, sub)],
                    gsems[b],
                )

        def wait_gather(j, b):
            for h in range(NSPLIT):
                pltpu.make_async_copy(
                    proj_hbm.at[idx_v.at[j, pl.ds(h * sub, sub)]],
                    bufs[b].at[pl.ds(h * sub, sub)],
                    gsems[b],
                ).wait()

        for b in range(NBUF):
            start_gather(b, b)

        def body(t, carry):
            i = t * NBUF
            for b in range(NBUF):
                j = i + b
                dst = out_hbm.at[j * NW + wid]
                wait_gather(j, b)
                pltpu.async_copy(bufs[b], dst, wsems[b])

                @pl.when(j + NBUF < NCH)
                def _():
                    pltpu.make_async_copy(bufs[b], dst, wsems[b]).wait()
                    start_gather(j + NBUF, b)

            return carry

        lax.fori_loop(0, NCH // NBUF, body, jnp.int32(0))

        for b in range(NBUF):
            j = NCH - NBUF + b
            pltpu.make_async_copy(bufs[b], out_hbm.at[j * NW + wid], wsems[b]).wait()

    return _gather


def kernel(src, emb1_weight, emb2_weight):
    proj = _project(emb1_weight.T, emb2_weight.T)
    idx = src.T.reshape(HIST, NW, BPW).transpose(1, 0, 2)
    out = _build_gather()(proj, idx)
    return out.reshape(HIST, BATCH, HID_DIM).transpose(1, 0, 2)


# speedup vs baseline: 9.5363x; 1.1043x over previous
import functools

import jax
import jax.numpy as jnp
from jax import lax
from jax.experimental import pallas as pl
from jax.experimental.pallas import tpu as pltpu
from jax.experimental.pallas import tpu_sc as plsc

INPUT_DIM = 100000
LAYER1_DIM = 64
HID_DIM = 128
BATCH = 4096
HIST = 50

NW = 32
BPW = BATCH // NW
NCH = HIST
NBUF = 5
NSPLIT = 2

COLS_BLK = 8192


def _proj_body(w1t_ref, w2t_ref, out_ref):
    out_ref[...] = lax.dot_general(
        w1t_ref[...], w2t_ref[...],
        dimension_numbers=(((0,), (0,)), ((), ())),
        preferred_element_type=jnp.float32,
    )


def _project(emb1_t, emb2_t):
    return pl.pallas_call(
        _proj_body,
        grid=(pl.cdiv(INPUT_DIM, COLS_BLK),),
        in_specs=[
            pl.BlockSpec((LAYER1_DIM, COLS_BLK), lambda i: (0, i)),
            pl.BlockSpec((LAYER1_DIM, HID_DIM), lambda i: (0, 0)),
        ],
        out_specs=pl.BlockSpec((COLS_BLK, HID_DIM), lambda i: (i, 0)),
        out_shape=jax.ShapeDtypeStruct((INPUT_DIM, HID_DIM), jnp.float32),
    )(emb1_t, emb2_t)


@functools.lru_cache(maxsize=1)
def _build_gather():
    mesh = plsc.VectorSubcoreMesh(core_axis_name="c", subcore_axis_name="s")

    @functools.partial(
        pl.kernel,
        mesh=mesh,
        out_type=jax.ShapeDtypeStruct((HIST * NW, BPW, HID_DIM), jnp.float32),
        scratch_types=[
            pltpu.VMEM((NCH, BPW), jnp.int32),
            *[pltpu.VMEM((BPW, HID_DIM), jnp.float32) for _ in range(NBUF)],
            *[pltpu.SemaphoreType.DMA for _ in range(2 * NBUF)],
        ],
    )
    def _gather(proj_hbm, idx_hbm, out_hbm, idx_v, *scratch):
        bufs = scratch[:NBUF]
        gsems = scratch[NBUF:2 * NBUF]
        wsems = scratch[2 * NBUF:]
        wid = lax.axis_index("s") * 2 + lax.axis_index("c")
        pltpu.sync_copy(idx_hbm.at[:, pl.ds(wid * BPW, BPW)], idx_v)
        sub = BPW // NSPLIT

        def start_gather(j, b):
            for h in range(NSPLIT):
                pltpu.async_copy(
                    proj_hbm.at[idx_v.at[j, pl.ds(h * sub, sub)]],
                    bufs[b].at[pl.ds(h * sub, sub)],
                    gsems[b],
                )

        def wait_gather(j, b):
            for h in range(NSPLIT):
                pltpu.make_async_copy(
                    proj_hbm.at[idx_v.at[j, pl.ds(h * sub, sub)]],
                    bufs[b].at[pl.ds(h * sub, sub)],
                    gsems[b],
                ).wait()

        for b in range(NBUF):
            start_gather(b, b)

        def body(t, carry):
            i = t * NBUF
            for b in range(NBUF):
                j = i + b
                dst = out_hbm.at[j * NW + wid]
                wait_gather(j, b)
                pltpu.async_copy(bufs[b], dst, wsems[b])

                @pl.when(j + NBUF < NCH)
                def _():
                    pltpu.make_async_copy(bufs[b], dst, wsems[b]).wait()
                    start_gather(j + NBUF, b)

            return carry

        lax.fori_loop(0, NCH // NBUF, body, jnp.int32(0))

        for b in range(NBUF):
            j = NCH - NBUF + b
            pltpu.make_async_copy(bufs[b], out_hbm.at[j * NW + wid], wsems[b]).wait()

    return _gather


def kernel(src, emb1_weight, emb2_weight):
    proj = _project(emb1_weight.T, emb2_weight.T)
    out = _build_gather()(proj, src.T)
    return out.reshape(HIST, BATCH, HID_DIM).transpose(1, 0, 2)


# speedup vs baseline: 9.6388x; 1.0108x over previous
import functools

import jax
import jax.numpy as jnp
from jax import lax
from jax.experimental import pallas as pl
from jax.experimental.pallas import tpu as pltpu
from jax.experimental.pallas import tpu_sc as plsc

INPUT_DIM = 100000
LAYER1_DIM = 64
HID_DIM = 128
BATCH = 4096
HIST = 50

NW = 32
BPW = BATCH // NW
NCH = HIST
NBUF = 5
NSPLIT = 2

COLS_BLK = 12544


def _proj_body(w1t_ref, w2t_ref, out_ref):
    out_ref[...] = lax.dot_general(
        w1t_ref[...], w2t_ref[...],
        dimension_numbers=(((0,), (0,)), ((), ())),
        preferred_element_type=jnp.float32,
    )


def _project(emb1_t, emb2_t):
    return pl.pallas_call(
        _proj_body,
        grid=(pl.cdiv(INPUT_DIM, COLS_BLK),),
        in_specs=[
            pl.BlockSpec((LAYER1_DIM, COLS_BLK), lambda i: (0, i)),
            pl.BlockSpec((LAYER1_DIM, HID_DIM), lambda i: (0, 0)),
        ],
        out_specs=pl.BlockSpec((COLS_BLK, HID_DIM), lambda i: (i, 0)),
        out_shape=jax.ShapeDtypeStruct((INPUT_DIM, HID_DIM), jnp.float32),
    )(emb1_t, emb2_t)


@functools.lru_cache(maxsize=1)
def _build_gather():
    mesh = plsc.VectorSubcoreMesh(core_axis_name="c", subcore_axis_name="s")

    @functools.partial(
        pl.kernel,
        mesh=mesh,
        out_type=jax.ShapeDtypeStruct((HIST * NW, BPW, HID_DIM), jnp.float32),
        scratch_types=[
            pltpu.VMEM((NCH, BPW), jnp.int32),
            *[pltpu.VMEM((BPW, HID_DIM), jnp.float32) for _ in range(NBUF)],
            *[pltpu.SemaphoreType.DMA for _ in range(2 * NBUF)],
        ],
    )
    def _gather(proj_hbm, idx_hbm, out_hbm, idx_v, *scratch):
        bufs = scratch[:NBUF]
        gsems = scratch[NBUF:2 * NBUF]
        wsems = scratch[2 * NBUF:]
        wid = lax.axis_index("s") * 2 + lax.axis_index("c")
        pltpu.sync_copy(idx_hbm.at[:, pl.ds(wid * BPW, BPW)], idx_v)
        sub = BPW // NSPLIT

        def start_gather(j, b):
            for h in range(NSPLIT):
                pltpu.async_copy(
                    proj_hbm.at[idx_v.at[j, pl.ds(h * sub, sub)]],
                    bufs[b].at[pl.ds(h * sub, sub)],
                    gsems[b],
                )

        def wait_gather(j, b):
            for h in range(NSPLIT):
                pltpu.make_async_copy(
                    proj_hbm.at[idx_v.at[j, pl.ds(h * sub, sub)]],
                    bufs[b].at[pl.ds(h * sub, sub)],
                    gsems[b],
                ).wait()

        for b in range(NBUF):
            start_gather(b, b)

        def wait_writes(j, b):
            for h in range(NSPLIT):
                pltpu.make_async_copy(
                    bufs[b].at[pl.ds(h * sub, sub)],
                    out_hbm.at[j * NW + wid].at[pl.ds(h * sub, sub)],
                    wsems[b],
                ).wait()

        def body(t, carry):
            i = t * NBUF
            for b in range(NBUF):
                j = i + b
                dst = out_hbm.at[j * NW + wid]
                for h in range(NSPLIT):
                    pltpu.make_async_copy(
                        proj_hbm.at[idx_v.at[j, pl.ds(h * sub, sub)]],
                        bufs[b].at[pl.ds(h * sub, sub)],
                        gsems[b],
                    ).wait()
                    pltpu.async_copy(
                        bufs[b].at[pl.ds(h * sub, sub)],
                        dst.at[pl.ds(h * sub, sub)],
                        wsems[b],
                    )

                @pl.when(j + NBUF < NCH)
                def _():
                    wait_writes(j, b)
                    start_gather(j + NBUF, b)

            return carry

        lax.fori_loop(0, NCH // NBUF, body, jnp.int32(0))

        for b in range(NBUF):
            wait_writes(NCH - NBUF + b, b)

    return _gather


def kernel(src, emb1_weight, emb2_weight):
    proj = _project(emb1_weight.T, emb2_weight.T)
    out = _build_gather()(proj, src.T)
    return out.reshape(HIST, BATCH, HID_DIM).transpose(1, 0, 2)
